# baseline (speedup 1.0000x reference)
import functools

import jax
import jax.numpy as jnp
from jax import lax
from jax.experimental import pallas as pl
from jax.experimental.pallas import tpu as pltpu
from jax.experimental.pallas import tpu_sc as plsc

VOCAB = 1000000
SEQ_LEN = 200
DIM = 64
BATCH = 4096

NUM_CORES = 2
NUM_SUBCORES = 16
NUM_WORKERS = NUM_CORES * NUM_SUBCORES
N_ROWS = BATCH * SEQ_LEN
ROWS_PER_WORKER = N_ROWS // NUM_WORKERS
SEQS_PER_WORKER = BATCH // NUM_WORKERS


@functools.partial(
    pl.kernel,
    mesh=plsc.VectorSubcoreMesh(core_axis_name="c", subcore_axis_name="s"),
    out_type=jax.ShapeDtypeStruct((N_ROWS, DIM), jnp.float32),
    compiler_params=pltpu.CompilerParams(use_tc_tiling_on_sc=False),
    scratch_types=[
        pltpu.VMEM((ROWS_PER_WORKER,), jnp.int32),
        pltpu.VMEM((SEQ_LEN, DIM), jnp.float32),
        pltpu.VMEM((SEQ_LEN, DIM), jnp.float32),
        pltpu.SemaphoreType.DMA,
        pltpu.SemaphoreType.DMA,
    ],
)
def _embed_kernel(tok_hbm, table_hbm, pe_hbm, out_hbm,
                  idx_all, pe_v, rows_v, sem_a, sem_b):
    wid = lax.axis_index("s") * NUM_CORES + lax.axis_index("c")
    base = wid * ROWS_PER_WORKER

    pltpu.sync_copy(tok_hbm.at[pl.ds(base, ROWS_PER_WORKER)], idx_all)
    pltpu.sync_copy(pe_hbm, pe_v)

    def per_seq(c, _):
        rb = c * SEQ_LEN
        g_a = pltpu.async_copy(
            table_hbm.at[idx_all.at[pl.ds(rb, 128)]],
            rows_v.at[pl.ds(0, 128)], sem_a)
        g_b = pltpu.async_copy(
            table_hbm.at[idx_all.at[pl.ds(rb + 128, SEQ_LEN - 128)]],
            rows_v.at[pl.ds(128, SEQ_LEN - 128)], sem_b)
        g_a.wait()
        g_b.wait()

        def add_pe(r, _):
            for j in range(DIM // 16):
                sl = pl.ds(j * 16, 16)
                rows_v[r, sl] = rows_v[r, sl] + pe_v[r, sl]
            return _

        lax.fori_loop(0, SEQ_LEN, add_pe, None)
        pltpu.sync_copy(rows_v, out_hbm.at[pl.ds(base + rb, SEQ_LEN)])
        return _

    lax.fori_loop(0, SEQS_PER_WORKER, per_seq, None)


def kernel(tokenize, table, pe):
    tok_flat = tokenize.reshape(-1).astype(jnp.int32)
    out = _embed_kernel(tok_flat, table, pe)
    return out.reshape(BATCH, SEQ_LEN, DIM)


# speedup vs baseline: 1.1488x; 1.1488x over previous
import functools

import jax
import jax.numpy as jnp
from jax import lax
from jax.experimental import pallas as pl
from jax.experimental.pallas import tpu as pltpu
from jax.experimental.pallas import tpu_sc as plsc

VOCAB = 1000000
SEQ_LEN = 200
DIM = 64
BATCH = 4096

NUM_CORES = 2
NUM_SUBCORES = 16
NUM_WORKERS = NUM_CORES * NUM_SUBCORES
N_ROWS = BATCH * SEQ_LEN
ROWS_PER_WORKER = N_ROWS // NUM_WORKERS
NCHUNKS = BATCH // NUM_WORKERS
NBUF = 4
G0 = 128
G1 = SEQ_LEN - G0


def _fire_gather(table_hbm, idx_all, buf, sem, rb):
    pltpu.async_copy(table_hbm.at[idx_all.at[pl.ds(rb, G0)]],
                     buf.at[pl.ds(0, G0)], sem)
    pltpu.async_copy(table_hbm.at[idx_all.at[pl.ds(rb + G0, G1)]],
                     buf.at[pl.ds(G0, G1)], sem)


def _wait_gather(table_hbm, idx_all, buf, sem, rb):
    pltpu.make_async_copy(table_hbm.at[idx_all.at[pl.ds(rb, G0)]],
                          buf.at[pl.ds(0, G0)], sem).wait()
    pltpu.make_async_copy(table_hbm.at[idx_all.at[pl.ds(rb + G0, G1)]],
                          buf.at[pl.ds(G0, G1)], sem).wait()


def _fire_store(buf, out_hbm, sem, gb):
    pltpu.async_copy(buf, out_hbm.at[pl.ds(gb, SEQ_LEN)], sem)


def _wait_store(buf, out_hbm, sem, gb):
    pltpu.make_async_copy(buf, out_hbm.at[pl.ds(gb, SEQ_LEN)], sem).wait()


def _add_pe(buf, pe_v):
    @plsc.parallel_loop(0, SEQ_LEN, step=1, unroll=8)
    def _(r):
        for j in range(DIM // 16):
            sl = pl.ds(j * 16, 16)
            buf[r, sl] = buf[r, sl] + pe_v[r, sl]


@functools.partial(
    pl.kernel,
    mesh=plsc.VectorSubcoreMesh(core_axis_name="c", subcore_axis_name="s"),
    out_type=jax.ShapeDtypeStruct((N_ROWS, DIM), jnp.float32),
    compiler_params=pltpu.CompilerParams(use_tc_tiling_on_sc=False),
    scratch_types=[
        pltpu.VMEM((ROWS_PER_WORKER,), jnp.int32),
        pltpu.VMEM((SEQ_LEN, DIM), jnp.float32),
        pltpu.VMEM((SEQ_LEN, DIM), jnp.float32),
        pltpu.VMEM((SEQ_LEN, DIM), jnp.float32),
        pltpu.VMEM((SEQ_LEN, DIM), jnp.float32),
        pltpu.VMEM((SEQ_LEN, DIM), jnp.float32),
        pltpu.SemaphoreType.DMA,
        pltpu.SemaphoreType.DMA,
        pltpu.SemaphoreType.DMA,
        pltpu.SemaphoreType.DMA,
        pltpu.SemaphoreType.DMA,
        pltpu.SemaphoreType.DMA,
        pltpu.SemaphoreType.DMA,
        pltpu.SemaphoreType.DMA,
    ],
)
def _embed_kernel(tok_hbm, table_hbm, pe_hbm, out_hbm,
                  idx_all, pe_v, buf0, buf1, buf2, buf3,
                  g0, g1, g2, g3, s0, s1, s2, s3):
    bufs = [buf0, buf1, buf2, buf3]
    gsems = [g0, g1, g2, g3]
    ssems = [s0, s1, s2, s3]
    wid = lax.axis_index("s") * NUM_CORES + lax.axis_index("c")
    base = wid * ROWS_PER_WORKER

    pltpu.sync_copy(tok_hbm.at[pl.ds(base, ROWS_PER_WORKER)], idx_all)
    pltpu.sync_copy(pe_hbm, pe_v)

    def step(c, b, do_store_wait, do_fire):
        rb = c * SEQ_LEN
        b2 = (b + 2) % NBUF
        if do_store_wait:
            _wait_store(bufs[b2], out_hbm, ssems[b2],
                        base + (c - 2) * SEQ_LEN)
        if do_fire:
            _fire_gather(table_hbm, idx_all, bufs[b2], gsems[b2],
                         (c + 2) * SEQ_LEN)
        _wait_gather(table_hbm, idx_all, bufs[b], gsems[b], rb)
        _add_pe(bufs[b], pe_v)
        _fire_store(bufs[b], out_hbm, ssems[b], base + rb)

    _fire_gather(table_hbm, idx_all, bufs[0], gsems[0], 0)
    _fire_gather(table_hbm, idx_all, bufs[1], gsems[1], SEQ_LEN)

    for b in range(NBUF):
        step(jnp.int32(b), b, do_store_wait=(b >= 2), do_fire=True)

    def group(g, _):
        for b in range(NBUF):
            step(g * NBUF + b, b, do_store_wait=True, do_fire=True)
        return _

    lax.fori_loop(1, NCHUNKS // NBUF - 1, group, None)

    gt = NCHUNKS - NBUF
    for b in range(NBUF):
        step(jnp.int32(gt + b), b, do_store_wait=True, do_fire=(b < 2))

    _wait_store(bufs[2], out_hbm, ssems[2], base + (NCHUNKS - 2) * SEQ_LEN)
    _wait_store(bufs[3], out_hbm, ssems[3], base + (NCHUNKS - 1) * SEQ_LEN)


def kernel(tokenize, table, pe):
    tok_flat = tokenize.reshape(-1).astype(jnp.int32)
    out = _embed_kernel(tok_flat, table, pe)
    return out.reshape(BATCH, SEQ_LEN, DIM)


# speedup vs baseline: 1.1516x; 1.0024x over previous
import functools

import jax
import jax.numpy as jnp
from jax import lax
from jax.experimental import pallas as pl
from jax.experimental.pallas import tpu as pltpu
from jax.experimental.pallas import tpu_sc as plsc

VOCAB = 1000000
SEQ_LEN = 200
DIM = 64
BATCH = 4096

NUM_CORES = 2
NUM_SUBCORES = 16
NUM_WORKERS = NUM_CORES * NUM_SUBCORES
NCHUNKS = BATCH // NUM_WORKERS
NBUF = 4
G0 = 128
G1 = SEQ_LEN - G0


def _fire_gather(table_hbm, idx_all, buf, sem, c):
    pltpu.async_copy(table_hbm.at[idx_all.at[c, pl.ds(0, G0)]],
                     buf.at[pl.ds(0, G0)], sem)
    pltpu.async_copy(table_hbm.at[idx_all.at[c, pl.ds(G0, G1)]],
                     buf.at[pl.ds(G0, G1)], sem)


def _wait_gather(table_hbm, idx_all, buf, sem, c):
    pltpu.make_async_copy(table_hbm.at[idx_all.at[c, pl.ds(0, G0)]],
                          buf.at[pl.ds(0, G0)], sem).wait()
    pltpu.make_async_copy(table_hbm.at[idx_all.at[c, pl.ds(G0, G1)]],
                          buf.at[pl.ds(G0, G1)], sem).wait()


def _fire_store(buf, out_hbm, sem, seq):
    pltpu.async_copy(buf, out_hbm.at[seq], sem)


def _wait_store(buf, out_hbm, sem, seq):
    pltpu.make_async_copy(buf, out_hbm.at[seq], sem).wait()


def _add_pe(buf, pe_v):
    @plsc.parallel_loop(0, SEQ_LEN, step=1, unroll=8)
    def _(r):
        for j in range(DIM // 16):
            sl = pl.ds(j * 16, 16)
            buf[r, sl] = buf[r, sl] + pe_v[r, sl]


@functools.partial(
    pl.kernel,
    mesh=plsc.VectorSubcoreMesh(core_axis_name="c", subcore_axis_name="s"),
    out_type=jax.ShapeDtypeStruct((BATCH, SEQ_LEN, DIM), jnp.float32),
    compiler_params=pltpu.CompilerParams(use_tc_tiling_on_sc=False),
    scratch_types=[
        pltpu.VMEM((NCHUNKS, SEQ_LEN), jnp.int32),
        pltpu.VMEM((SEQ_LEN, DIM), jnp.float32),
        pltpu.VMEM((SEQ_LEN, DIM), jnp.float32),
        pltpu.VMEM((SEQ_LEN, DIM), jnp.float32),
        pltpu.VMEM((SEQ_LEN, DIM), jnp.float32),
        pltpu.VMEM((SEQ_LEN, DIM), jnp.float32),
        pltpu.SemaphoreType.DMA,
        pltpu.SemaphoreType.DMA,
        pltpu.SemaphoreType.DMA,
        pltpu.SemaphoreType.DMA,
        pltpu.SemaphoreType.DMA,
        pltpu.SemaphoreType.DMA,
        pltpu.SemaphoreType.DMA,
        pltpu.SemaphoreType.DMA,
    ],
)
def _embed_kernel(tok_hbm, table_hbm, pe_hbm, out_hbm,
                  idx_all, pe_v, buf0, buf1, buf2, buf3,
                  g0, g1, g2, g3, s0, s1, s2, s3):
    bufs = [buf0, buf1, buf2, buf3]
    gsems = [g0, g1, g2, g3]
    ssems = [s0, s1, s2, s3]
    wid = lax.axis_index("s") * NUM_CORES + lax.axis_index("c")
    seq0 = wid * NCHUNKS

    pltpu.sync_copy(tok_hbm.at[pl.ds(seq0, NCHUNKS)], idx_all)
    pltpu.sync_copy(pe_hbm, pe_v)

    def step(c, b, do_store_wait, do_fire):
        b2 = (b + 2) % NBUF
        if do_store_wait:
            _wait_store(bufs[b2], out_hbm, ssems[b2], seq0 + c - 2)
        if do_fire:
            _fire_gather(table_hbm, idx_all, bufs[b2], gsems[b2], c + 2)
        _wait_gather(table_hbm, idx_all, bufs[b], gsems[b], c)
        _add_pe(bufs[b], pe_v)
        _fire_store(bufs[b], out_hbm, ssems[b], seq0 + c)

    _fire_gather(table_hbm, idx_all, bufs[0], gsems[0], 0)
    _fire_gather(table_hbm, idx_all, bufs[1], gsems[1], 1)

    for b in range(NBUF):
        step(jnp.int32(b), b, do_store_wait=(b >= 2), do_fire=True)

    def group(g, _):
        for b in range(NBUF):
            step(g * NBUF + b, b, do_store_wait=True, do_fire=True)
        return _

    lax.fori_loop(1, NCHUNKS // NBUF - 1, group, None)

    gt = NCHUNKS - NBUF
    for b in range(NBUF):
        step(jnp.int32(gt + b), b, do_store_wait=True, do_fire=(b < 2))

    _wait_store(bufs[2], out_hbm, ssems[2], seq0 + NCHUNKS - 2)
    _wait_store(bufs[3], out_hbm, ssems[3], seq0 + NCHUNKS - 1)


def kernel(tokenize, table, pe):
    return _embed_kernel(tokenize.astype(jnp.int32), table, pe)


# speedup vs baseline: 1.2167x; 1.0565x over previous
import functools

import jax
import jax.numpy as jnp
from jax import lax
from jax.experimental import pallas as pl
from jax.experimental.pallas import tpu as pltpu
from jax.experimental.pallas import tpu_sc as plsc

VOCAB = 1000000
SEQ_LEN = 200
DIM = 64
BATCH = 4096

NUM_CORES = 2
NUM_SUBCORES = 16
NUM_WORKERS = NUM_CORES * NUM_SUBCORES
NCHUNKS = BATCH // NUM_WORKERS
G0 = 128
G1 = SEQ_LEN - G0


def _fire_idx(tok_hbm, idx, sem, seq):
    pltpu.async_copy(tok_hbm.at[pl.ds(seq, 1)], idx, sem)


def _wait_idx(tok_hbm, idx, sem, seq):
    pltpu.make_async_copy(tok_hbm.at[pl.ds(seq, 1)], idx, sem).wait()


def _compute_idxh(idx, idxh):
    for j in range(12):
        sl = pl.ds(j * 16, 16)
        idxh[0, sl] = jax.lax.shift_right_logical(idx[0, sl], 1)
    sl_t = pl.ds(SEQ_LEN - 16, 16)
    idxh[0, sl_t] = jax.lax.shift_right_logical(idx[0, sl_t], 1)


def _fire_gather(table_hbm, idxh, buf, sem):
    pltpu.async_copy(table_hbm.at[idxh.at[0, pl.ds(0, G0)]],
                     buf.at[pl.ds(0, G0)], sem)
    pltpu.async_copy(table_hbm.at[idxh.at[0, pl.ds(G0, G1)]],
                     buf.at[pl.ds(G0, G1)], sem)


def _wait_gather(table_hbm, idxh, buf, sem):
    pltpu.make_async_copy(table_hbm.at[idxh.at[0, pl.ds(0, G0)]],
                          buf.at[pl.ds(0, G0)], sem).wait()
    pltpu.make_async_copy(table_hbm.at[idxh.at[0, pl.ds(G0, G1)]],
                          buf.at[pl.ds(G0, G1)], sem).wait()


def _fire_store(cbuf, out_hbm, sem, seq):
    pltpu.async_copy(cbuf, out_hbm.at[pl.ds(seq * SEQ_LEN, SEQ_LEN)], sem)


def _wait_store(cbuf, out_hbm, sem, seq):
    pltpu.make_async_copy(
        cbuf, out_hbm.at[pl.ds(seq * SEQ_LEN, SEQ_LEN)], sem).wait()


def _add_rows(gbuf, cbuf, pe_v, r0, lanes, toks):
    offs = (toks & 1) * DIM
    for lane in lanes:
        r = r0 + lane
        off = offs[lane]
        pe_r = r0 // 2 + lane // 2
        pe_off = (lane % 2) * DIM
        for j in range(DIM // 16):
            src = gbuf[r, pl.ds(off + j * 16, 16)]
            pe_row = pe_v[pe_r, pl.ds(pe_off + j * 16, 16)]
            cbuf[r, pl.ds(j * 16, 16)] = src + pe_row


def _add_pe(gbuf, cbuf, pe_v, idx):
    @plsc.parallel_loop(0, SEQ_LEN // 16, step=1)
    def _(g):
        r0 = g * 16
        toks = idx[0, pl.ds(r0, 16)]
        _add_rows(gbuf, cbuf, pe_v, r0, range(16), toks)
    toks_t = idx[0, pl.ds(SEQ_LEN - 16, 16)]
    _add_rows(gbuf, cbuf, pe_v, SEQ_LEN - 16, range(8, 16), toks_t)


@functools.partial(
    pl.kernel,
    mesh=plsc.VectorSubcoreMesh(core_axis_name="c", subcore_axis_name="s"),
    out_type=jax.ShapeDtypeStruct((BATCH * SEQ_LEN, DIM), jnp.float32),
    compiler_params=pltpu.CompilerParams(use_tc_tiling_on_sc=True),
    scratch_types=(
        [pltpu.VMEM((SEQ_LEN // 2, 2 * DIM), jnp.float32)]
        + [pltpu.VMEM((SEQ_LEN, 2 * DIM), jnp.float32)] * 2
        + [pltpu.VMEM((SEQ_LEN, DIM), jnp.float32)] * 2
        + [pltpu.VMEM((1, SEQ_LEN), jnp.int32)] * 2
        + [pltpu.VMEM((1, SEQ_LEN), jnp.int32)] * 2
        + [pltpu.SemaphoreType.DMA] * 6
    ),
)
def _embed_kernel(tok_hbm, table_hbm, pe_hbm, out_hbm, pe_v,
                  gb0, gb1, cb0, cb1, i0, i1, h0, h1,
                  si0, si1, sg0, sg1, ss0, ss1):
    gbufs = [gb0, gb1]
    cbufs = [cb0, cb1]
    idxs = [i0, i1]
    idxhs = [h0, h1]
    isems = [si0, si1]
    gsems = [sg0, sg1]
    ssems = [ss0, ss1]
    wid = lax.axis_index("s") * NUM_CORES + lax.axis_index("c")
    seq0 = wid * NCHUNKS

    pltpu.sync_copy(pe_hbm, pe_v)

    def fire_idx(c, b):
        _fire_idx(tok_hbm, idxs[b], isems[b], seq0 + c)

    def fire_gather(c, b):
        _wait_idx(tok_hbm, idxs[b], isems[b], seq0 + c)
        _compute_idxh(idxs[b], idxhs[b])
        _fire_gather(table_hbm, idxhs[b], gbufs[b], gsems[b])

    def step(c, b, do_store_wait, do_fire_idx, do_fire_gather):
        b1 = 1 - b
        if do_fire_gather:
            fire_gather(c + 1, b1)
        _wait_gather(table_hbm, idxhs[b], gbufs[b], gsems[b])
        if do_store_wait:
            _wait_store(cbufs[b], out_hbm, ssems[b], seq0 + c - 2)
        _add_pe(gbufs[b], cbufs[b], pe_v, idxs[b])
        _fire_store(cbufs[b], out_hbm, ssems[b], seq0 + c)
        if do_fire_idx:
            fire_idx(c + 2, b)

    fire_idx(0, 0)
    fire_idx(1, 1)
    fire_gather(0, 0)

    step(jnp.int32(0), 0, do_store_wait=False,
         do_fire_idx=True, do_fire_gather=True)
    step(jnp.int32(1), 1, do_store_wait=False,
         do_fire_idx=True, do_fire_gather=True)

    def group(g, _):
        for b in range(2):
            step(g * 2 + b, b, do_store_wait=True,
                 do_fire_idx=True, do_fire_gather=True)
        return _

    lax.fori_loop(1, NCHUNKS // 2 - 1, group, None)

    step(jnp.int32(NCHUNKS - 2), 0, do_store_wait=True,
         do_fire_idx=False, do_fire_gather=True)
    step(jnp.int32(NCHUNKS - 1), 1, do_store_wait=True,
         do_fire_idx=False, do_fire_gather=False)

    _wait_store(cbufs[0], out_hbm, ssems[0], seq0 + NCHUNKS - 2)
    _wait_store(cbufs[1], out_hbm, ssems[1], seq0 + NCHUNKS - 1)


def kernel(tokenize, table, pe):
    table2 = table.reshape(VOCAB // 2, 2 * DIM)
    pe2 = pe.reshape(SEQ_LEN // 2, 2 * DIM)
    out = _embed_kernel(tokenize.astype(jnp.int32), table2, pe2)
    return out.reshape(BATCH, SEQ_LEN, DIM)


# speedup vs baseline: 1.7243x; 1.4173x over previous
import functools

import jax
import jax.numpy as jnp
from jax import lax
from jax.experimental import pallas as pl
from jax.experimental.pallas import tpu as pltpu
from jax.experimental.pallas import tpu_sc as plsc

VOCAB = 1000000
SEQ_LEN = 200
DIM = 64
BATCH = 4096

NUM_CORES = 2
NUM_SUBCORES = 16
NUM_WORKERS = NUM_CORES * NUM_SUBCORES
NCHUNKS = BATCH // NUM_WORKERS
NBUF = 4


def _fire_idx(tok_hbm, idxv, sem, seq):
    pltpu.async_copy(tok_hbm.at[pl.ds(seq, 1)], idxv, sem)


def _wait_idx(tok_hbm, idxv, sem, seq):
    pltpu.make_async_copy(tok_hbm.at[pl.ds(seq, 1)], idxv, sem).wait()


def _enqueue_rows(table_hbm, gbuf, sem, r0, lanes, toks):
    for lane in lanes:
        tok = toks[lane]
        pltpu.async_copy(table_hbm.at[pl.ds(tok, 1)],
                         gbuf.at[pl.ds(r0 + lane, 1)], sem)


def _fire_gather(table_hbm, idxv, gbuf, sem):
    def row_group(g, _):
        toks = idxv[0, pl.ds(g * 16, 16)]
        _enqueue_rows(table_hbm, gbuf, sem, g * 16, range(16), toks)
        return _
    lax.fori_loop(0, SEQ_LEN // 16, row_group, None)
    toks_t = idxv[0, pl.ds(SEQ_LEN - 16, 16)]
    _enqueue_rows(table_hbm, gbuf, sem, SEQ_LEN - 16, range(8, 16), toks_t)


def _wait_gather(table_hbm, gbuf, sem):
    pltpu.make_async_copy(table_hbm.at[pl.ds(0, SEQ_LEN)], gbuf, sem).wait()


def _fire_store(gbuf, out_hbm, sem, seq):
    pltpu.async_copy(gbuf, out_hbm.at[pl.ds(seq * SEQ_LEN, SEQ_LEN)], sem)


def _wait_store(gbuf, out_hbm, sem, seq):
    pltpu.make_async_copy(
        gbuf, out_hbm.at[pl.ds(seq * SEQ_LEN, SEQ_LEN)], sem).wait()


def _add_pe(gbuf, pe_v):
    @plsc.parallel_loop(0, SEQ_LEN, step=1, unroll=8)
    def _(r):
        pe_off = (r % 2) * DIM
        for j in range(DIM // 16):
            sl = pl.ds(j * 16, 16)
            pe_row = pe_v[r // 2, pl.ds(pe_off + j * 16, 16)]
            gbuf[r, sl] = gbuf[r, sl] + pe_row


@functools.partial(
    pl.kernel,
    mesh=plsc.VectorSubcoreMesh(core_axis_name="c", subcore_axis_name="s"),
    out_type=jax.ShapeDtypeStruct((BATCH * SEQ_LEN, DIM), jnp.float32),
    compiler_params=pltpu.CompilerParams(use_tc_tiling_on_sc=True),
    scratch_types=(
        [pltpu.VMEM((SEQ_LEN // 2, 2 * DIM), jnp.float32)]
        + [pltpu.VMEM((SEQ_LEN, DIM), jnp.float32)] * NBUF
        + [pltpu.VMEM((1, SEQ_LEN), jnp.int32)] * NBUF
        + [pltpu.SemaphoreType.DMA] * (3 * NBUF)
    ),
)
def _embed_kernel(tok_hbm, table_hbm, pe_hbm, out_hbm, pe_v,
                  b0, b1, b2, b3, m0, m1, m2, m3,
                  si0, si1, si2, si3, g0, g1, g2, g3, s0, s1, s2, s3):
    gbufs = [b0, b1, b2, b3]
    idxvs = [m0, m1, m2, m3]
    isems = [si0, si1, si2, si3]
    gsems = [g0, g1, g2, g3]
    ssems = [s0, s1, s2, s3]
    wid = lax.axis_index("s") * NUM_CORES + lax.axis_index("c")
    seq0 = wid * NCHUNKS

    pltpu.sync_copy(pe_hbm, pe_v)

    def fire_idx(c, b):
        _fire_idx(tok_hbm, idxvs[b], isems[b], seq0 + c)

    def fire_gather(c, b):
        _wait_idx(tok_hbm, idxvs[b], isems[b], seq0 + c)
        _fire_gather(table_hbm, idxvs[b], gbufs[b], gsems[b])

    def step(c, b, do_store_wait, do_fire_idx, do_fire_gather):
        b2 = (b + 2) % NBUF
        b3 = (b + 3) % NBUF
        if do_store_wait:
            _wait_store(gbufs[b2], out_hbm, ssems[b2], seq0 + c - 2)
        if do_fire_idx:
            fire_idx(c + 3, b3)
        if do_fire_gather:
            fire_gather(c + 2, b2)
        _wait_gather(table_hbm, gbufs[b], gsems[b])
        _add_pe(gbufs[b], pe_v)
        _fire_store(gbufs[b], out_hbm, ssems[b], seq0 + c)

    fire_idx(0, 0)
    fire_idx(1, 1)
    fire_idx(2, 2)
    fire_gather(0, 0)
    fire_gather(1, 1)

    for b in range(NBUF):
        step(jnp.int32(b), b, do_store_wait=(b >= 2),
             do_fire_idx=True, do_fire_gather=True)

    def group(g, _):
        for b in range(NBUF):
            step(g * NBUF + b, b, do_store_wait=True,
                 do_fire_idx=True, do_fire_gather=True)
        return _

    lax.fori_loop(1, NCHUNKS // NBUF - 1, group, None)

    gt = NCHUNKS - NBUF
    for b in range(NBUF):
        step(jnp.int32(gt + b), b, do_store_wait=True,
             do_fire_idx=(b < 1), do_fire_gather=(b < 2))

    _wait_store(gbufs[2], out_hbm, ssems[2], seq0 + NCHUNKS - 2)
    _wait_store(gbufs[3], out_hbm, ssems[3], seq0 + NCHUNKS - 1)


def kernel(tokenize, table, pe):
    pe2 = pe.reshape(SEQ_LEN // 2, 2 * DIM)
    out = _embed_kernel(tokenize.astype(jnp.int32), table, pe2)
    return out.reshape(BATCH, SEQ_LEN, DIM)
